# baseline (speedup 1.0000x reference)
import functools

import jax
import jax.numpy as jnp
from jax import lax
from jax.experimental import pallas as pl
from jax.experimental.pallas import tpu as pltpu
from jax.experimental.pallas import tpu_sc as plsc

_VOCAB = 1000000
_DIM = 64
_BATCH = 16384
_HIST = 200
_NW = 32
_BPW = _BATCH // _NW
_NB = 4
_LO = 104
_HI = _HIST - _LO
_NITER = _BPW // (2 * _NB)

_mesh = plsc.VectorSubcoreMesh(core_axis_name="c", subcore_axis_name="s")


@functools.partial(
    pl.kernel,
    mesh=_mesh,
    out_type=jax.ShapeDtypeStruct((_BATCH, _LO, 2 * _DIM), jnp.float32),
    scratch_types=[
        pltpu.VMEM((2 * _NB, _HIST), jnp.int32),
        pltpu.VMEM((_NB, _LO, _DIM), jnp.float32),
        pltpu.VMEM((_NB, _HI, _DIM), jnp.float32),
        pltpu.VMEM((_NB, _LO, _DIM), jnp.float32),
        pltpu.VMEM((_NB, _HI, _DIM), jnp.float32),
        pltpu.SemaphoreType.DMA,
        pltpu.SemaphoreType.DMA,
        pltpu.SemaphoreType.DMA,
        pltpu.SemaphoreType.DMA,
    ],
    compiler_params=pltpu.CompilerParams(use_tc_tiling_on_sc=False),
)
def _embed_gather(x_hbm, table_hbm, out_hbm, idx_v, lo_a, hi_a, lo_b, hi_b,
                  sem_ga, sem_gb, sem_wa, sem_wb):
    wid = lax.axis_index("s") * 2 + lax.axis_index("c")
    b_base = wid * _BPW

    def fire_gathers(lo_buf, hi_buf, j0, sem):
        copies = []
        for j in range(_NB):
            copies.append(
                pltpu.async_copy(
                    table_hbm.at[idx_v.at[j0 + j, pl.ds(0, _LO)]],
                    lo_buf.at[j], sem))
            copies.append(
                pltpu.async_copy(
                    table_hbm.at[idx_v.at[j0 + j, pl.ds(_LO, _HI)]],
                    hi_buf.at[j], sem))
        return copies

    def out_slices(b0):
        lo = out_hbm.at[pl.ds(b0, _NB), :, pl.ds(0, _DIM)]
        hi = out_hbm.at[pl.ds(b0, _NB), pl.ds(0, _HI), pl.ds(_DIM, _DIM)]
        return lo, hi

    def body(g, carry):
        b0 = b_base + g * 2 * _NB
        pltpu.sync_copy(x_hbm.at[pl.ds(b0, 2 * _NB)], idx_v)

        out_a_lo, out_a_hi = out_slices(b0)
        out_b_lo, out_b_hi = out_slices(b0 + _NB)

        @pl.when(g > 0)
        def _():
            pltpu.make_async_copy(lo_a, out_a_lo, sem_wa).wait()
            pltpu.make_async_copy(hi_a, out_a_hi, sem_wa).wait()

        ga = fire_gathers(lo_a, hi_a, 0, sem_ga)

        @pl.when(g > 0)
        def _():
            pltpu.make_async_copy(lo_b, out_b_lo, sem_wb).wait()
            pltpu.make_async_copy(hi_b, out_b_hi, sem_wb).wait()

        gb = fire_gathers(lo_b, hi_b, _NB, sem_gb)

        for c in ga:
            c.wait()
        pltpu.async_copy(lo_a, out_a_lo, sem_wa)
        pltpu.async_copy(hi_a, out_a_hi, sem_wa)
        for c in gb:
            c.wait()
        pltpu.async_copy(lo_b, out_b_lo, sem_wb)
        pltpu.async_copy(hi_b, out_b_hi, sem_wb)
        return carry

    lax.fori_loop(0, _NITER, body, 0)

    last = b_base + (_NITER - 1) * 2 * _NB
    la_lo, la_hi = out_slices(last)
    lb_lo, lb_hi = out_slices(last + _NB)
    pltpu.make_async_copy(lo_a, la_lo, sem_wa).wait()
    pltpu.make_async_copy(hi_a, la_hi, sem_wa).wait()
    pltpu.make_async_copy(lo_b, lb_lo, sem_wb).wait()
    pltpu.make_async_copy(hi_b, lb_hi, sem_wb).wait()


def kernel(x, table):
    packed = _embed_gather(x, table)
    return jnp.concatenate(
        [packed[:, :, :_DIM], packed[:, :_HI, _DIM:]], axis=1)


# speedup vs baseline: 1.0755x; 1.0755x over previous
import functools

import jax
import jax.numpy as jnp
from jax import lax
from jax.experimental import pallas as pl
from jax.experimental.pallas import tpu as pltpu
from jax.experimental.pallas import tpu_sc as plsc

_VOCAB = 1000000
_DIM = 64
_BATCH = 16384
_HIST = 200
_NW = 32
_BPW = _BATCH // _NW
_NB = 4
_LO = 104
_HI = _HIST - _LO
_NITER = _BPW // (2 * _NB)

_mesh = plsc.VectorSubcoreMesh(core_axis_name="c", subcore_axis_name="s")


@functools.partial(
    pl.kernel,
    mesh=_mesh,
    out_type=jax.ShapeDtypeStruct((_BATCH, _LO, 2 * _DIM), jnp.float32),
    scratch_types=[
        pltpu.VMEM((2 * _NB, _HIST), jnp.int32),
        pltpu.VMEM((_NB, _LO, _DIM), jnp.float32),
        pltpu.VMEM((_NB, _HI, _DIM), jnp.float32),
        pltpu.VMEM((_NB, _LO, _DIM), jnp.float32),
        pltpu.VMEM((_NB, _HI, _DIM), jnp.float32),
        pltpu.SemaphoreType.DMA,
        pltpu.SemaphoreType.DMA,
        pltpu.SemaphoreType.DMA,
        pltpu.SemaphoreType.DMA,
    ],
    compiler_params=pltpu.CompilerParams(use_tc_tiling_on_sc=False),
)
def _embed_gather(x_hbm, table_hbm, out_hbm, idx_v, lo_a, hi_a, lo_b, hi_b,
                  sem_ga, sem_gb, sem_wa, sem_wb):
    wid = lax.axis_index("s") * 2 + lax.axis_index("c")
    b_base = wid * _BPW

    def fire_gathers(lo_buf, hi_buf, j0, sem):
        copies = []
        for j in range(_NB):
            copies.append(
                pltpu.async_copy(
                    table_hbm.at[idx_v.at[j0 + j, pl.ds(0, _LO)]],
                    lo_buf.at[j], sem))
            copies.append(
                pltpu.async_copy(
                    table_hbm.at[idx_v.at[j0 + j, pl.ds(_LO, _HI)]],
                    hi_buf.at[j], sem))
        return copies

    def out_slices(b0):
        lo = out_hbm.at[pl.ds(b0, _NB), :, pl.ds(0, _DIM)]
        hi = out_hbm.at[pl.ds(b0, _NB), pl.ds(0, _HI), pl.ds(_DIM, _DIM)]
        return lo, hi

    def body(g, carry):
        b0 = b_base + g * 2 * _NB
        pltpu.sync_copy(x_hbm.at[pl.ds(b0, 2 * _NB)], idx_v)

        out_a_lo, out_a_hi = out_slices(b0)
        out_b_lo, out_b_hi = out_slices(b0 + _NB)

        @pl.when(g > 0)
        def _():
            pltpu.make_async_copy(lo_a, out_a_lo, sem_wa).wait()
            pltpu.make_async_copy(hi_a, out_a_hi, sem_wa).wait()

        ga = fire_gathers(lo_a, hi_a, 0, sem_ga)

        @pl.when(g > 0)
        def _():
            pltpu.make_async_copy(lo_b, out_b_lo, sem_wb).wait()
            pltpu.make_async_copy(hi_b, out_b_hi, sem_wb).wait()

        gb = fire_gathers(lo_b, hi_b, _NB, sem_gb)

        for c in ga:
            c.wait()
        pltpu.async_copy(lo_a, out_a_lo, sem_wa)
        pltpu.async_copy(hi_a, out_a_hi, sem_wa)
        for c in gb:
            c.wait()
        pltpu.async_copy(lo_b, out_b_lo, sem_wb)
        pltpu.async_copy(hi_b, out_b_hi, sem_wb)
        return carry

    lax.fori_loop(0, _NITER, body, 0)

    last = b_base + (_NITER - 1) * 2 * _NB
    la_lo, la_hi = out_slices(last)
    lb_lo, lb_hi = out_slices(last + _NB)
    pltpu.make_async_copy(lo_a, la_lo, sem_wa).wait()
    pltpu.make_async_copy(hi_a, la_hi, sem_wa).wait()
    pltpu.make_async_copy(lo_b, lb_lo, sem_wb).wait()
    pltpu.make_async_copy(hi_b, lb_hi, sem_wb).wait()


_TCB = 32


def _unpack_body(in_ref, out_ref):
    blk = in_ref[...]
    out_ref[:, 0:_LO, :] = blk[:, :, 0:_DIM]
    out_ref[:, _LO:_HIST, :] = blk[:, 0:_HI, _DIM:2 * _DIM]


_unpack = pl.pallas_call(
    _unpack_body,
    out_shape=jax.ShapeDtypeStruct((_BATCH, _HIST, _DIM), jnp.float32),
    grid=(_BATCH // _TCB,),
    in_specs=[pl.BlockSpec((_TCB, _LO, 2 * _DIM), lambda i: (i, 0, 0))],
    out_specs=pl.BlockSpec((_TCB, _HIST, _DIM), lambda i: (i, 0, 0)),
)


def kernel(x, table):
    packed = _embed_gather(x, table)
    return _unpack(packed)


# speedup vs baseline: 1.3240x; 1.2311x over previous
import functools

import jax
import jax.numpy as jnp
from jax import lax
from jax.experimental import pallas as pl
from jax.experimental.pallas import tpu as pltpu
from jax.experimental.pallas import tpu_sc as plsc

_VOCAB = 1000000
_DIM = 64
_BATCH = 16384
_HIST = 200
_HALF = _HIST // 2
_NW = 32
_BPW = _BATCH // _NW
_NB = 4
_NITER = _BPW // (2 * _NB)

_mesh = plsc.VectorSubcoreMesh(core_axis_name="c", subcore_axis_name="s")


@functools.partial(
    pl.kernel,
    mesh=_mesh,
    out_type=jax.ShapeDtypeStruct((_BATCH, _HALF, 2 * _DIM), jnp.float32),
    scratch_types=[
        pltpu.VMEM((2 * _NB, 2, _HALF), jnp.int32),
        pltpu.VMEM((_NB, _HALF, _DIM), jnp.float32),
        pltpu.VMEM((_NB, _HALF, _DIM), jnp.float32),
        pltpu.VMEM((_NB, _HALF, _DIM), jnp.float32),
        pltpu.VMEM((_NB, _HALF, _DIM), jnp.float32),
        pltpu.SemaphoreType.DMA,
        pltpu.SemaphoreType.DMA,
        pltpu.SemaphoreType.DMA,
        pltpu.SemaphoreType.DMA,
    ],
    compiler_params=pltpu.CompilerParams(use_tc_tiling_on_sc=False),
)
def _embed_gather(x2_hbm, table_hbm, out_hbm, idx_v, ev_a, od_a, ev_b, od_b,
                  sem_ga, sem_gb, sem_wa, sem_wb):
    wid = lax.axis_index("s") * 2 + lax.axis_index("c")
    b_base = wid * _BPW

    def fire_gathers(ev_buf, od_buf, j0, sem):
        copies = []
        for j in range(_NB):
            copies.append(
                pltpu.async_copy(
                    table_hbm.at[idx_v.at[j0 + j, 0]], ev_buf.at[j], sem))
            copies.append(
                pltpu.async_copy(
                    table_hbm.at[idx_v.at[j0 + j, 1]], od_buf.at[j], sem))
        return copies

    def out_slices(b0):
        ev = out_hbm.at[pl.ds(b0, _NB), :, pl.ds(0, _DIM)]
        od = out_hbm.at[pl.ds(b0, _NB), :, pl.ds(_DIM, _DIM)]
        return ev, od

    def body(g, carry):
        b0 = b_base + g * 2 * _NB
        pltpu.sync_copy(x2_hbm.at[pl.ds(b0, 2 * _NB)], idx_v)

        out_a_ev, out_a_od = out_slices(b0)
        out_b_ev, out_b_od = out_slices(b0 + _NB)

        @pl.when(g > 0)
        def _():
            pltpu.make_async_copy(ev_a, out_a_ev, sem_wa).wait()
            pltpu.make_async_copy(od_a, out_a_od, sem_wa).wait()

        ga = fire_gathers(ev_a, od_a, 0, sem_ga)

        @pl.when(g > 0)
        def _():
            pltpu.make_async_copy(ev_b, out_b_ev, sem_wb).wait()
            pltpu.make_async_copy(od_b, out_b_od, sem_wb).wait()

        gb = fire_gathers(ev_b, od_b, _NB, sem_gb)

        for c in ga:
            c.wait()
        pltpu.async_copy(ev_a, out_a_ev, sem_wa)
        pltpu.async_copy(od_a, out_a_od, sem_wa)
        for c in gb:
            c.wait()
        pltpu.async_copy(ev_b, out_b_ev, sem_wb)
        pltpu.async_copy(od_b, out_b_od, sem_wb)
        return carry

    lax.fori_loop(0, _NITER, body, 0)

    last = b_base + (_NITER - 1) * 2 * _NB
    la_ev, la_od = out_slices(last)
    lb_ev, lb_od = out_slices(last + _NB)
    pltpu.make_async_copy(ev_a, la_ev, sem_wa).wait()
    pltpu.make_async_copy(od_a, la_od, sem_wa).wait()
    pltpu.make_async_copy(ev_b, lb_ev, sem_wb).wait()
    pltpu.make_async_copy(od_b, lb_od, sem_wb).wait()


def kernel(x, table):
    x3 = x.reshape(_BATCH, _HALF, 2)
    x2 = jnp.stack([x3[:, :, 0], x3[:, :, 1]], axis=1)
    packed = _embed_gather(x2, table)
    return packed.reshape(_BATCH, _HIST, _DIM)


# speedup vs baseline: 1.8008x; 1.3601x over previous
import functools

import jax
import jax.numpy as jnp
from jax import lax
from jax.experimental import pallas as pl
from jax.experimental.pallas import tpu as pltpu
from jax.experimental.pallas import tpu_sc as plsc

_VOCAB = 1000000
_DIM = 64
_BATCH = 16384
_HIST = 200
_NW = 32
_BPW = _BATCH // _NW
_NB = 4
_SPLITS = ((0, 104), (104, 96))
_NITER = _BPW // (2 * _NB)

_mesh = plsc.VectorSubcoreMesh(core_axis_name="c", subcore_axis_name="s")


@functools.partial(
    pl.kernel,
    mesh=_mesh,
    out_type=jax.ShapeDtypeStruct((_BATCH, _HIST, 2 * _DIM), jnp.float32),
    scratch_types=[
        pltpu.VMEM((2 * _NB, _HIST), jnp.int32),
        pltpu.VMEM((_NB, _HIST, _DIM), jnp.float32),
        pltpu.VMEM((_NB, _HIST, _DIM), jnp.float32),
        pltpu.SemaphoreType.DMA,
        pltpu.SemaphoreType.DMA,
        pltpu.SemaphoreType.DMA,
        pltpu.SemaphoreType.DMA,
    ],
    compiler_params=pltpu.CompilerParams(use_tc_tiling_on_sc=False),
)
def _embed_gather(x_hbm, table_hbm, out_hbm, idx_v, rows_a, rows_b,
                  sem_ga, sem_gb, sem_wa, sem_wb):
    wid = lax.axis_index("s") * 2 + lax.axis_index("c")
    b_base = wid * _BPW

    def fire_gathers(rows_buf, j0, sem):
        copies = []
        for j in range(_NB):
            for h, w in _SPLITS:
                copies.append(
                    pltpu.async_copy(
                        table_hbm.at[idx_v.at[j0 + j, pl.ds(h, w)]],
                        rows_buf.at[j, pl.ds(h, w)],
                        sem,
                    )
                )
        return copies

    def out_slice(b0):
        return out_hbm.at[pl.ds(b0, _NB), :, pl.ds(0, _DIM)]

    def body(g, carry):
        b0 = b_base + g * 2 * _NB
        pltpu.sync_copy(x_hbm.at[pl.ds(b0, 2 * _NB)], idx_v)

        out_a = out_slice(b0)
        out_b = out_slice(b0 + _NB)

        @pl.when(g > 0)
        def _():
            pltpu.make_async_copy(rows_a, out_a, sem_wa).wait()

        ga = fire_gathers(rows_a, 0, sem_ga)

        @pl.when(g > 0)
        def _():
            pltpu.make_async_copy(rows_b, out_b, sem_wb).wait()

        gb = fire_gathers(rows_b, _NB, sem_gb)

        for c in ga:
            c.wait()
        pltpu.async_copy(rows_a, out_a, sem_wa)
        for c in gb:
            c.wait()
        pltpu.async_copy(rows_b, out_b, sem_wb)
        return carry

    lax.fori_loop(0, _NITER, body, 0)

    last = b_base + (_NITER - 1) * 2 * _NB
    pltpu.make_async_copy(rows_a, out_slice(last), sem_wa).wait()
    pltpu.make_async_copy(rows_b, out_slice(last + _NB), sem_wb).wait()


def kernel(x, table):
    padded = _embed_gather(x, table)
    return padded[:, :, :_DIM]
